# baseline (speedup 1.0000x reference)
import jax
import jax.numpy as jnp
from jax.experimental import pallas as pl
from jax.experimental.pallas import tpu as pltpu

_EPS = 1e-6
_TB = 256


def _fused_body(h0_ref, h1_ref, h2_ref, h3_ref, w_ref, o_ref):
    red = (h0_ref[...] + h1_ref[...]) + (h2_ref[...] + h3_ref[...])
    var = jnp.sum(red * red, axis=-1, keepdims=True) * (1.0 / red.shape[-1])
    o_ref[...] = red * jax.lax.rsqrt(var + _EPS) * w_ref[...]


def kernel(hidden_states, residual, weight):
    del residual
    tp, tokens, hidden = hidden_states.shape
    w2 = weight.reshape(1, hidden)
    parts = [hidden_states[i] for i in range(tp)]
    tok_spec = pl.BlockSpec((_TB, hidden), lambda i: (i, 0))
    out = pl.pallas_call(
        _fused_body,
        grid=(tokens // _TB,),
        in_specs=[tok_spec] * tp + [pl.BlockSpec((1, hidden), lambda i: (0, 0))],
        out_specs=tok_spec,
        out_shape=jax.ShapeDtypeStruct((tokens, hidden), hidden_states.dtype),
        compiler_params=pltpu.CompilerParams(
            dimension_semantics=("parallel",),
            vmem_limit_bytes=50 * 1024 * 1024,
        ),
    )(*parts, w2)
    return out


# speedup vs baseline: 2.3701x; 2.3701x over previous
import jax
import jax.numpy as jnp
from jax.experimental import pallas as pl
from jax.experimental.pallas import tpu as pltpu

_EPS = 1e-6
_TB = 512


def _body(h_ref, w_ref, o_ref, acc_ref):
    k = pl.program_id(1)
    ntp = pl.num_programs(1)

    @pl.when(k == 0)
    def _init():
        acc_ref[...] = h_ref[0]

    @pl.when(k > 0)
    def _acc():
        acc_ref[...] = acc_ref[...] + h_ref[0]

    @pl.when(k == ntp - 1)
    def _finish():
        red = acc_ref[...]
        var = jnp.sum(red * red, axis=-1, keepdims=True) * (1.0 / red.shape[-1])
        o_ref[...] = red * jax.lax.rsqrt(var + _EPS) * w_ref[...]


def kernel(hidden_states, residual, weight):
    del residual
    tp, tokens, hidden = hidden_states.shape
    w2 = weight.reshape(1, hidden)
    out = pl.pallas_call(
        _body,
        grid=(tokens // _TB, tp),
        in_specs=[
            pl.BlockSpec((1, _TB, hidden), lambda i, k: (k, i, 0)),
            pl.BlockSpec((1, hidden), lambda i, k: (0, 0)),
        ],
        out_specs=pl.BlockSpec((_TB, hidden), lambda i, k: (i, 0)),
        out_shape=jax.ShapeDtypeStruct((tokens, hidden), hidden_states.dtype),
        scratch_shapes=[pltpu.VMEM((_TB, hidden), jnp.float32)],
        compiler_params=pltpu.CompilerParams(
            dimension_semantics=("parallel", "arbitrary"),
            vmem_limit_bytes=50 * 1024 * 1024,
        ),
    )(hidden_states, w2)
    return out


# speedup vs baseline: 2.6514x; 1.1187x over previous
import jax
import jax.numpy as jnp
from jax.experimental import pallas as pl
from jax.experimental.pallas import tpu as pltpu

_EPS = 1e-6
_TB = 256


def _fused_body(h_ref, w_ref, o_ref):
    h = h_ref[...]
    red = (h[0] + h[1]) + (h[2] + h[3])
    var = jnp.sum(red * red, axis=-1, keepdims=True) * (1.0 / h.shape[-1])
    o_ref[...] = red * jax.lax.rsqrt(var + _EPS) * w_ref[...]


def kernel(hidden_states, residual, weight):
    del residual
    tp, tokens, hidden = hidden_states.shape
    w2 = weight.reshape(1, hidden)
    out = pl.pallas_call(
        _fused_body,
        grid=(tokens // _TB,),
        in_specs=[
            pl.BlockSpec((tp, _TB, hidden), lambda i: (0, i, 0)),
            pl.BlockSpec((1, hidden), lambda i: (0, 0)),
        ],
        out_specs=pl.BlockSpec((_TB, hidden), lambda i: (i, 0)),
        out_shape=jax.ShapeDtypeStruct((tokens, hidden), hidden_states.dtype),
        compiler_params=pltpu.CompilerParams(
            dimension_semantics=("parallel",),
            vmem_limit_bytes=50 * 1024 * 1024,
        ),
    )(hidden_states, w2)
    return out
